# Initial kernel scaffold; baseline (speedup 1.0000x reference)
import jax
import jax.numpy as jnp
from jax.experimental import pallas as pl


def kernel(hidden_states, W_router, W_gate, W_up, W_down):
    raise NotImplementedError("write your pallas kernel here")


# speedup vs baseline: 2.6588x; 2.6588x over previous
import jax
import jax.numpy as jnp
from jax.experimental import pallas as pl
from jax.experimental.pallas import tpu as pltpu

E = 16
TOP_K = 2
H = 768
I = 256


def _moe_body(x_ref, wr_ref, wg_ref, wu_ref, wd_ref, out_ref,
              w1_ref, w2_ref, i1_ref, i2_ref):
    e = pl.program_id(0)

    @pl.when(e == 0)
    def _router():
        x = x_ref[...]
        logits = jax.lax.dot_general(
            x, wr_ref[...], (((1,), (1,)), ((), ())),
            preferred_element_type=jnp.float32)
        scores = jax.nn.sigmoid(logits)
        eidx = jax.lax.broadcasted_iota(jnp.int32, scores.shape, 1)
        m1 = jnp.max(scores, axis=1, keepdims=True)
        is1 = scores == m1
        i1 = jnp.min(jnp.where(is1, eidx, E), axis=1, keepdims=True)
        excl = eidx == i1
        s2 = jnp.where(excl, -jnp.inf, scores)
        m2 = jnp.max(s2, axis=1, keepdims=True)
        i2 = jnp.min(jnp.where(s2 == m2, eidx, E), axis=1, keepdims=True)
        denom = m1 + m2 + 1e-20
        w1_ref[...] = m1 / denom
        w2_ref[...] = m2 / denom
        i1_ref[...] = i1
        i2_ref[...] = i2

    x = x_ref[...]
    g = jax.lax.dot_general(x, wg_ref[0], (((1,), (0,)), ((), ())),
                            preferred_element_type=jnp.float32)
    u = jax.lax.dot_general(x, wu_ref[0], (((1,), (0,)), ((), ())),
                            preferred_element_type=jnp.float32)
    hmid = g * jax.nn.sigmoid(g) * u
    d = jax.lax.dot_general(hmid, wd_ref[0], (((1,), (0,)), ((), ())),
                            preferred_element_type=jnp.float32)
    w_e = (jnp.where(i1_ref[...] == e, w1_ref[...], 0.0) +
           jnp.where(i2_ref[...] == e, w2_ref[...], 0.0))
    contrib = d * w_e

    @pl.when(e == 0)
    def _init():
        out_ref[...] = contrib

    @pl.when(e != 0)
    def _acc():
        out_ref[...] += contrib


def kernel(hidden_states, W_router, W_gate, W_up, W_down):
    b, s, h = hidden_states.shape
    T = b * s
    x = hidden_states.reshape(T, h)
    out = pl.pallas_call(
        _moe_body,
        grid=(E,),
        in_specs=[
            pl.BlockSpec((T, H), lambda e: (0, 0)),
            pl.BlockSpec((E, H), lambda e: (0, 0)),
            pl.BlockSpec((1, H, I), lambda e: (e, 0, 0)),
            pl.BlockSpec((1, H, I), lambda e: (e, 0, 0)),
            pl.BlockSpec((1, I, H), lambda e: (e, 0, 0)),
        ],
        out_specs=pl.BlockSpec((T, H), lambda e: (0, 0)),
        out_shape=jax.ShapeDtypeStruct((T, H), jnp.float32),
        scratch_shapes=[
            pltpu.VMEM((T, 1), jnp.float32),
            pltpu.VMEM((T, 1), jnp.float32),
            pltpu.VMEM((T, 1), jnp.int32),
            pltpu.VMEM((T, 1), jnp.int32),
        ],
    )(x, W_router, W_gate, W_up, W_down)
    return out.reshape(b, s, h)
